# baseline (speedup 1.0000x reference)
import functools

import jax
import jax.numpy as jnp
from jax import lax
from jax.experimental import pallas as pl
from jax.experimental.pallas import tpu as pltpu
from jax.experimental.pallas import tpu_sc as plsc

NC = 2
NS = 16
NW = NC * NS
L = 16

CHUNK = 128
KCOLS = 512


def _pack_kernel(v, d):
    npair = 128 * ((v + 255) // 256)

    def body(xt_ref, out_ref):
        blk = xt_ref[...]
        lo = jnp.transpose(blk[:, :128], (1, 0))
        hi = jnp.transpose(blk[:, 128:], (1, 0))
        out_ref[...] = jnp.concatenate([lo, hi], axis=1)

    return pl.pallas_call(
        body,
        grid=(npair // 128,),
        in_specs=[pl.BlockSpec((d, 256), lambda j: (0, j))],
        out_specs=pl.BlockSpec((128, 2 * d), lambda j: (j, 0)),
        out_shape=jax.ShapeDtypeStruct((npair, 2 * d), jnp.float32),
    )


def _untranspose_kernel(b, d):
    grid = b // KCOLS

    def body(in_ref, out_ref):
        out_ref[...] = jnp.transpose(in_ref[...], (1, 0))

    return pl.pallas_call(
        body,
        grid=(grid,),
        in_specs=[pl.BlockSpec((KCOLS, d), lambda j: (j, 0))],
        out_specs=pl.BlockSpec((d, KCOLS), lambda j: (0, j)),
        out_shape=jax.ShapeDtypeStruct((d, b), jnp.float32),
    )


def _make_gather(B, D):
    n_per_w = B // NW
    n_chunks = n_per_w // CHUNK
    mesh = plsc.VectorSubcoreMesh(core_axis_name="c", subcore_axis_name="s")

    @functools.partial(
        pl.kernel,
        mesh=mesh,
        out_type=jax.ShapeDtypeStruct((B, D), jnp.float32),
        compiler_params=pltpu.CompilerParams(needs_layout_passes=False),
        scratch_types=[
            pltpu.VMEM((n_per_w,), jnp.int32),
            pltpu.VMEM((2, CHUNK), jnp.int32),
            pltpu.VMEM((2, CHUNK, 2 * D), jnp.float32),
            pltpu.VMEM((2, CHUNK, D), jnp.float32),
            pltpu.SemaphoreType.DMA,
            pltpu.SemaphoreType.DMA,
            pltpu.SemaphoreType.DMA,
            pltpu.SemaphoreType.DMA,
        ],
    )
    def k(xp_hbm, idx_hbm, out_hbm, idx_v, pidx_v, pairs_v, rows_v, *sems):
        g0, g1, s0, s1 = sems
        gsems = (g0, g1)
        ssems = (s0, s1)
        wid = lax.axis_index("s") * NC + lax.axis_index("c")
        row0 = wid * n_per_w
        pltpu.sync_copy(idx_hbm.at[pl.ds(row0, n_per_w)], idx_v)

        def fire(c, b):
            for g in range(CHUNK // L):
                vec = idx_v[pl.ds(c * CHUNK + g * L, L)]
                prow = jax.lax.shift_left(
                    jax.lax.shift_right_logical(vec, 8), 7)
                prow = prow + jax.lax.bitwise_and(vec, 127)
                pidx_v[b, pl.ds(g * L, L)] = prow
            pltpu.async_copy(xp_hbm.at[pidx_v.at[b]], pairs_v.at[b],
                             gsems[b])

        def drain_gather(b):
            pltpu.make_async_copy(
                xp_hbm.at[pl.ds(0, CHUNK)], pairs_v.at[b], gsems[b]).wait()

        def select_store(c, b):
            def group(g, carry):
                vec = idx_v[pl.ds(c * CHUNK + g * L, L)]
                offs = jax.lax.bitwise_and(
                    jax.lax.shift_right_logical(vec, 7), 1) * D
                for j in range(L):
                    o = lax.squeeze(lax.slice(offs, (j,), (j + 1,)),
                                    dimensions=(0,))
                    r = g * L + j
                    for jb in range(D // L):
                        col = o + jb * L
                        v = plsc.load_gather(
                            pairs_v.at[b],
                            [jnp.full((L,), r, jnp.int32),
                             col + lax.iota(jnp.int32, L)])
                        rows_v[b, r, pl.ds(jb * L, L)] = v
                return carry

            lax.fori_loop(0, CHUNK // L, group, 0)
            pltpu.async_copy(
                rows_v.at[b],
                out_hbm.at[pl.ds(row0 + c * CHUNK, CHUNK)], ssems[b])

        def drain_store(b):
            pltpu.make_async_copy(
                rows_v.at[b], out_hbm.at[pl.ds(0, CHUNK)], ssems[b]).wait()

        fire(0, 0)

        def body(i, carry):
            c = i * 2
            for b in range(2):

                @pl.when(c + b + 1 < n_chunks)
                def _():
                    fire(c + b + 1, 1 - b)

                drain_gather(b)

                @pl.when(c + b >= 2)
                def _():
                    drain_store(b)

                select_store(c + b, b)
            return carry

        lax.fori_loop(0, n_chunks // 2, body, 0)
        drain_store(0)
        drain_store(1)

    return k


def kernel(x, dim, idx):
    B = idx.shape[0]
    V, D = x.shape
    idx32 = (idx + dim).astype(jnp.int32)
    xp = _pack_kernel(V, D)(x.T)
    outp = _make_gather(B, D)(xp, idx32)
    return _untranspose_kernel(B, D)(outp).T


# speedup vs baseline: 1.7251x; 1.7251x over previous
import functools

import jax
import jax.numpy as jnp
from jax import lax
from jax.experimental import pallas as pl
from jax.experimental.pallas import tpu as pltpu
from jax.experimental.pallas import tpu_sc as plsc

NC = 2
NS = 16
NW = NC * NS
L = 16

CHUNK = 128
KCOLS = 512


def _mxu_t(a, d):
    eye = jnp.eye(d, dtype=jnp.float32)
    return jax.lax.dot_general(
        a, eye, (((0,), (0,)), ((), ())),
        precision=jax.lax.Precision.HIGHEST,
        preferred_element_type=jnp.float32)


def _pack_kernel(v, d):
    groups_per_step = 4
    cols = 256 * groups_per_step
    grid = (v + cols - 1) // cols
    npair = grid * 128 * groups_per_step

    def body(xt_ref, out_ref):
        blk = xt_ref[...]
        parts = []
        for g in range(groups_per_step):
            lo = _mxu_t(blk[:, g * 256:g * 256 + 128], d)
            hi = _mxu_t(blk[:, g * 256 + 128:g * 256 + 256], d)
            parts.append(jnp.concatenate([lo, hi], axis=1))
        out_ref[...] = jnp.concatenate(parts, axis=0)

    return pl.pallas_call(
        body,
        grid=(grid,),
        in_specs=[pl.BlockSpec((d, cols), lambda j: (0, j))],
        out_specs=pl.BlockSpec((128 * groups_per_step, 2 * d),
                               lambda j: (j, 0)),
        out_shape=jax.ShapeDtypeStruct((npair, 2 * d), jnp.float32),
    )


def _untranspose_kernel(b, d):
    grid = b // KCOLS

    def body(in_ref, out_ref):
        blk = in_ref[...]
        out_ref[...] = jnp.concatenate(
            [_mxu_t(blk[k * 128:(k + 1) * 128, :], 128)
             for k in range(KCOLS // 128)], axis=1)

    return pl.pallas_call(
        body,
        grid=(grid,),
        in_specs=[pl.BlockSpec((KCOLS, d), lambda j: (j, 0))],
        out_specs=pl.BlockSpec((d, KCOLS), lambda j: (0, j)),
        out_shape=jax.ShapeDtypeStruct((d, b), jnp.float32),
    )


def _make_gather(B, D):
    n_per_w = B // NW
    n_chunks = n_per_w // CHUNK
    mesh = plsc.VectorSubcoreMesh(core_axis_name="c", subcore_axis_name="s")

    @functools.partial(
        pl.kernel,
        mesh=mesh,
        out_type=jax.ShapeDtypeStruct((B, D), jnp.float32),
        compiler_params=pltpu.CompilerParams(needs_layout_passes=False),
        scratch_types=[
            pltpu.VMEM((n_per_w,), jnp.int32),
            pltpu.VMEM((2, CHUNK), jnp.int32),
            pltpu.VMEM((2, CHUNK, 2 * D), jnp.float32),
            pltpu.VMEM((2, CHUNK, D), jnp.float32),
            pltpu.SemaphoreType.DMA,
            pltpu.SemaphoreType.DMA,
            pltpu.SemaphoreType.DMA,
            pltpu.SemaphoreType.DMA,
        ],
    )
    def k(xp_hbm, idx_hbm, out_hbm, idx_v, pidx_v, pairs_v, rows_v, *sems):
        g0, g1, s0, s1 = sems
        gsems = (g0, g1)
        ssems = (s0, s1)
        wid = lax.axis_index("s") * NC + lax.axis_index("c")
        row0 = wid * n_per_w
        pltpu.sync_copy(idx_hbm.at[pl.ds(row0, n_per_w)], idx_v)

        def fire(c, b):
            for g in range(CHUNK // L):
                vec = idx_v[pl.ds(c * CHUNK + g * L, L)]
                prow = jax.lax.shift_left(
                    jax.lax.shift_right_logical(vec, 8), 7)
                prow = prow + jax.lax.bitwise_and(vec, 127)
                pidx_v[b, pl.ds(g * L, L)] = prow
            pltpu.async_copy(xp_hbm.at[pidx_v.at[b]], pairs_v.at[b],
                             gsems[b])

        def drain_gather(b):
            pltpu.make_async_copy(
                xp_hbm.at[pl.ds(0, CHUNK)], pairs_v.at[b], gsems[b]).wait()

        def select_store(c, b):
            def group(g, carry):
                vec = idx_v[pl.ds(c * CHUNK + g * L, L)]
                offs = jax.lax.bitwise_and(
                    jax.lax.shift_right_logical(vec, 7), 1) * D
                for j in range(L):
                    o = lax.squeeze(lax.slice(offs, (j,), (j + 1,)),
                                    dimensions=(0,))
                    r = g * L + j
                    for jb in range(D // L):
                        col = o + jb * L
                        v = plsc.load_gather(
                            pairs_v.at[b],
                            [jnp.full((L,), r, jnp.int32),
                             col + lax.iota(jnp.int32, L)])
                        rows_v[b, r, pl.ds(jb * L, L)] = v
                return carry

            lax.fori_loop(0, CHUNK // L, group, 0)
            pltpu.async_copy(
                rows_v.at[b],
                out_hbm.at[pl.ds(row0 + c * CHUNK, CHUNK)], ssems[b])

        def drain_store(b):
            pltpu.make_async_copy(
                rows_v.at[b], out_hbm.at[pl.ds(0, CHUNK)], ssems[b]).wait()

        fire(0, 0)

        def body(i, carry):
            c = i * 2
            for b in range(2):

                @pl.when(c + b + 1 < n_chunks)
                def _():
                    fire(c + b + 1, 1 - b)

                drain_gather(b)

                @pl.when(c + b >= 2)
                def _():
                    drain_store(b)

                select_store(c + b, b)
            return carry

        lax.fori_loop(0, n_chunks // 2, body, 0)
        drain_store(0)
        drain_store(1)

    return k


def kernel(x, dim, idx):
    B = idx.shape[0]
    V, D = x.shape
    idx32 = (idx + dim).astype(jnp.int32)
    xp = _pack_kernel(V, D)(x.T)
    outp = _make_gather(B, D)(xp, idx32)
    return _untranspose_kernel(B, D)(outp).T


# speedup vs baseline: 3.1118x; 1.8039x over previous
import functools

import jax
import jax.numpy as jnp
from jax import lax
from jax.experimental import pallas as pl
from jax.experimental.pallas import tpu as pltpu
from jax.experimental.pallas import tpu_sc as plsc

NC = 2
NS = 16
NW = NC * NS
L = 16

CHUNK = 128
KCOLS = 2048


def _pack_kernel(v, d):
    grid = (v + KCOLS - 1) // KCOLS
    vpad = grid * KCOLS

    def body(xt_ref, eye_ref, out_ref):
        blk = xt_ref[...]
        t = jax.lax.dot_general(
            blk, eye_ref[...], (((0,), (0,)), ((), ())),
            precision=jax.lax.Precision.HIGHEST,
            preferred_element_type=jnp.float32)
        out_ref[...] = jnp.concatenate([t, t], axis=1)

    return pl.pallas_call(
        body,
        grid=(grid,),
        in_specs=[pl.BlockSpec((d, KCOLS), lambda j: (0, j)),
                  pl.BlockSpec((d, d), lambda j: (0, 0))],
        out_specs=pl.BlockSpec((KCOLS, 2 * d), lambda j: (j, 0)),
        out_shape=jax.ShapeDtypeStruct((vpad, 2 * d), jnp.float32),
    )


def _make_gather(B, D):
    n_per_w = B // NW
    n_chunks = n_per_w // CHUNK
    mesh = plsc.VectorSubcoreMesh(core_axis_name="c", subcore_axis_name="s")

    @functools.partial(
        pl.kernel,
        mesh=mesh,
        out_type=jax.ShapeDtypeStruct((B, D), jnp.float32),
        compiler_params=pltpu.CompilerParams(needs_layout_passes=False),
        scratch_types=[
            pltpu.VMEM((n_per_w,), jnp.int32),
            pltpu.VMEM((2, CHUNK, 2 * D), jnp.float32),
            pltpu.VMEM((2, CHUNK, D), jnp.float32),
            pltpu.SemaphoreType.DMA,
            pltpu.SemaphoreType.DMA,
            pltpu.SemaphoreType.DMA,
            pltpu.SemaphoreType.DMA,
        ],
    )
    def k(xp_hbm, idx_hbm, out_hbm, idx_v, pairs_v, rows_v, *sems):
        g0, g1, s0, s1 = sems
        gsems = (g0, g1)
        ssems = (s0, s1)
        wid = lax.axis_index("s") * NC + lax.axis_index("c")
        row0 = wid * n_per_w
        pltpu.sync_copy(idx_hbm.at[pl.ds(row0, n_per_w)], idx_v)

        def fire(c, b):
            pltpu.async_copy(
                xp_hbm.at[idx_v.at[pl.ds(c * CHUNK, CHUNK)]],
                pairs_v.at[b], gsems[b])

        def drain_gather(b):
            pltpu.make_async_copy(
                xp_hbm.at[pl.ds(0, CHUNK)], pairs_v.at[b], gsems[b]).wait()

        def select_store(c, b):
            def group(g, carry):
                for j in range(L):
                    r = g * L + j
                    for jb in range(D // L):
                        rows_v[b, r, pl.ds(jb * L, L)] = (
                            pairs_v[b, r, pl.ds(jb * L, L)])
                return carry

            lax.fori_loop(0, CHUNK // L, group, 0)
            pltpu.async_copy(
                rows_v.at[b],
                out_hbm.at[pl.ds(row0 + c * CHUNK, CHUNK)], ssems[b])

        def drain_store(b):
            pltpu.make_async_copy(
                rows_v.at[b], out_hbm.at[pl.ds(0, CHUNK)], ssems[b]).wait()

        fire(0, 0)

        def body(i, carry):
            c = i * 2
            for b in range(2):

                @pl.when(c + b + 1 < n_chunks)
                def _():
                    fire(c + b + 1, 1 - b)

                drain_gather(b)

                @pl.when(c + b >= 2)
                def _():
                    drain_store(b)

                select_store(c + b, b)
            return carry

        lax.fori_loop(0, n_chunks // 2, body, 0)
        drain_store(0)
        drain_store(1)

    return k


def kernel(x, dim, idx):
    B = idx.shape[0]
    V, D = x.shape
    idx32 = (idx + dim).astype(jnp.int32)
    xp = _pack_kernel(V, D)(x.T, jnp.eye(D, dtype=jnp.float32))
    return _make_gather(B, D)(xp, idx32)


# speedup vs baseline: 3.8389x; 1.2336x over previous
import functools

import jax
import jax.numpy as jnp
from jax import lax
from jax.experimental import pallas as pl
from jax.experimental.pallas import tpu as pltpu
from jax.experimental.pallas import tpu_sc as plsc

NC = 2
NS = 16
NW = NC * NS
L = 16

CHUNK = 128
KCOLS = 8192


def _pack_kernel(v, d):
    grid = (v + KCOLS - 1) // KCOLS
    vpad = grid * KCOLS

    def body(xt_ref, eye_ref, out_ref):
        blk = xt_ref[...]
        t = jax.lax.dot_general(
            blk, eye_ref[...], (((0,), (0,)), ((), ())),
            precision=jax.lax.Precision.HIGHEST,
            preferred_element_type=jnp.float32)
        out_ref[...] = jnp.concatenate([t, t], axis=1)

    return pl.pallas_call(
        body,
        grid=(grid,),
        in_specs=[pl.BlockSpec((d, KCOLS), lambda j: (0, j)),
                  pl.BlockSpec((d, d), lambda j: (0, 0))],
        out_specs=pl.BlockSpec((KCOLS, 2 * d), lambda j: (j, 0)),
        out_shape=jax.ShapeDtypeStruct((vpad, 2 * d), jnp.float32),
    )


def _make_gather(B, D):
    n_per_w = B // NW
    n_chunks = n_per_w // CHUNK
    mesh = plsc.VectorSubcoreMesh(core_axis_name="c", subcore_axis_name="s")

    @functools.partial(
        pl.kernel,
        mesh=mesh,
        out_type=jax.ShapeDtypeStruct((B, D), jnp.float32),
        compiler_params=pltpu.CompilerParams(needs_layout_passes=False),
        scratch_types=[
            pltpu.VMEM((n_per_w,), jnp.int32),
            pltpu.VMEM((2, CHUNK, 2 * D), jnp.float32),
            pltpu.VMEM((2, CHUNK, D), jnp.float32),
            pltpu.SemaphoreType.DMA,
            pltpu.SemaphoreType.DMA,
            pltpu.SemaphoreType.DMA,
            pltpu.SemaphoreType.DMA,
        ],
    )
    def k(xp_hbm, idx_hbm, out_hbm, idx_v, pairs_v, rows_v, *sems):
        g0, g1, s0, s1 = sems
        gsems = (g0, g1)
        ssems = (s0, s1)
        wid = lax.axis_index("s") * NC + lax.axis_index("c")
        row0 = wid * n_per_w
        pltpu.sync_copy(idx_hbm.at[pl.ds(row0, n_per_w)], idx_v)

        def fire(c, b):
            pltpu.async_copy(
                xp_hbm.at[idx_v.at[pl.ds(c * CHUNK, CHUNK)]],
                pairs_v.at[b], gsems[b])

        def drain_gather(b):
            pltpu.make_async_copy(
                xp_hbm.at[pl.ds(0, CHUNK)], pairs_v.at[b], gsems[b]).wait()

        def select_store(c, b):
            def group(g, carry):
                for j in range(L):
                    r = g * L + j
                    for jb in range(D // L):
                        rows_v[b, r, pl.ds(jb * L, L)] = (
                            pairs_v[b, r, pl.ds(jb * L, L)])
                return carry

            lax.fori_loop(0, CHUNK // L, group, 0)
            pltpu.async_copy(
                rows_v.at[b],
                out_hbm.at[pl.ds(row0 + c * CHUNK, CHUNK)], ssems[b])

        def drain_store(b):
            pltpu.make_async_copy(
                rows_v.at[b], out_hbm.at[pl.ds(0, CHUNK)], ssems[b]).wait()

        fire(0, 0)

        def body(i, carry):
            c = i * 2
            for b in range(2):

                @pl.when(c + b + 1 < n_chunks)
                def _():
                    fire(c + b + 1, 1 - b)

                drain_gather(b)

                @pl.when(c + b >= 2)
                def _():
                    drain_store(b)

                select_store(c + b, b)
            return carry

        lax.fori_loop(0, n_chunks // 2, body, 0)
        drain_store(0)
        drain_store(1)

    return k


def kernel(x, dim, idx):
    B = idx.shape[0]
    V, D = x.shape
    idx32 = (idx + dim).astype(jnp.int32)
    xp = _pack_kernel(V, D)(x.T, jnp.eye(D, dtype=jnp.float32))
    return _make_gather(B, D)(xp, idx32)


# speedup vs baseline: 3.8666x; 1.0072x over previous
import functools

import jax
import jax.numpy as jnp
from jax import lax
from jax.experimental import pallas as pl
from jax.experimental.pallas import tpu as pltpu
from jax.experimental.pallas import tpu_sc as plsc

NC = 2
NS = 16
NW = NC * NS
L = 16

CHUNK = 128
KCOLS = 12288


def _pack_kernel(v, d):
    grid = (v + KCOLS - 1) // KCOLS
    vpad = grid * KCOLS

    def body(xt_ref, eye_ref, out_ref):
        blk = xt_ref[...]
        t = jax.lax.dot_general(
            blk, eye_ref[...], (((0,), (0,)), ((), ())),
            precision=jax.lax.Precision.HIGHEST,
            preferred_element_type=jnp.float32)
        out_ref[...] = jnp.concatenate([t, t], axis=1)

    return pl.pallas_call(
        body,
        grid=(grid,),
        in_specs=[pl.BlockSpec((d, KCOLS), lambda j: (0, j)),
                  pl.BlockSpec((d, d), lambda j: (0, 0))],
        out_specs=pl.BlockSpec((KCOLS, 2 * d), lambda j: (j, 0)),
        out_shape=jax.ShapeDtypeStruct((vpad, 2 * d), jnp.float32),
    )


def _make_gather(B, D):
    n_per_w = B // NW
    n_chunks = n_per_w // CHUNK
    mesh = plsc.VectorSubcoreMesh(core_axis_name="c", subcore_axis_name="s")

    @functools.partial(
        pl.kernel,
        mesh=mesh,
        out_type=jax.ShapeDtypeStruct((B, D), jnp.float32),
        compiler_params=pltpu.CompilerParams(needs_layout_passes=False),
        scratch_types=[
            pltpu.VMEM((n_per_w,), jnp.int32),
            pltpu.VMEM((2, CHUNK, 2 * D), jnp.float32),
            pltpu.VMEM((2, CHUNK, D), jnp.float32),
            pltpu.SemaphoreType.DMA,
            pltpu.SemaphoreType.DMA,
            pltpu.SemaphoreType.DMA,
            pltpu.SemaphoreType.DMA,
        ],
    )
    def k(xp_hbm, idx_hbm, out_hbm, idx_v, pairs_v, rows_v, *sems):
        g0, g1, s0, s1 = sems
        gsems = (g0, g1)
        ssems = (s0, s1)
        wid = lax.axis_index("s") * NC + lax.axis_index("c")
        row0 = wid * n_per_w
        pltpu.sync_copy(idx_hbm.at[pl.ds(row0, n_per_w)], idx_v)

        def fire(c, b):
            pltpu.async_copy(
                xp_hbm.at[idx_v.at[pl.ds(c * CHUNK, CHUNK)]],
                pairs_v.at[b], gsems[b])

        def drain_gather(b):
            pltpu.make_async_copy(
                xp_hbm.at[pl.ds(0, CHUNK)], pairs_v.at[b], gsems[b]).wait()

        def select_store(c, b):
            def group(g, carry):
                for j in range(L):
                    r = g * L + j
                    for jb in range(D // L):
                        rows_v[b, r, pl.ds(jb * L, L)] = (
                            pairs_v[b, r, pl.ds(jb * L, L)])
                return carry

            lax.fori_loop(0, CHUNK // L, group, 0)
            pltpu.async_copy(
                rows_v.at[b],
                out_hbm.at[pl.ds(row0 + c * CHUNK, CHUNK)], ssems[b])

        def drain_store(b):
            pltpu.make_async_copy(
                rows_v.at[b], out_hbm.at[pl.ds(0, CHUNK)], ssems[b]).wait()

        fire(0, 0)

        def body(i, carry):
            c = i * 2
            for b in range(2):

                @pl.when(c + b + 1 < n_chunks)
                def _():
                    @pl.when(c + b >= 1)
                    def _():
                        drain_store(1 - b)

                    fire(c + b + 1, 1 - b)

                drain_gather(b)
                select_store(c + b, b)
            return carry

        lax.fori_loop(0, n_chunks // 2, body, 0)
        drain_store(0)
        drain_store(1)

    return k


def kernel(x, dim, idx):
    B = idx.shape[0]
    V, D = x.shape
    idx32 = (idx + dim).astype(jnp.int32)
    xp = _pack_kernel(V, D)(x.T, jnp.eye(D, dtype=jnp.float32))
    return _make_gather(B, D)(xp, idx32)


# speedup vs baseline: 4.3781x; 1.1323x over previous
import functools

import jax
import jax.numpy as jnp
from jax import lax
from jax.experimental import pallas as pl
from jax.experimental.pallas import tpu as pltpu
from jax.experimental.pallas import tpu_sc as plsc

NC = 2
NS = 16
NW = NC * NS
L = 16

CHUNK = 128
KCOLS = 12288


def _pack_kernel(v, d):
    grid = (v + KCOLS - 1) // KCOLS
    vpad = grid * KCOLS

    def body(xt_ref, eye_ref, out_ref):
        blk = xt_ref[...]
        eye = eye_ref[...]
        hi = blk.astype(jnp.bfloat16).astype(jnp.float32)
        lo = blk - hi
        dims = (((0,), (0,)), ((), ()))
        t = (jax.lax.dot_general(hi, eye, dims,
                                 preferred_element_type=jnp.float32)
             + jax.lax.dot_general(lo, eye, dims,
                                   preferred_element_type=jnp.float32))
        out_ref[...] = jnp.concatenate([t, t], axis=1)

    return pl.pallas_call(
        body,
        grid=(grid,),
        in_specs=[pl.BlockSpec((d, KCOLS), lambda j: (0, j)),
                  pl.BlockSpec((d, d), lambda j: (0, 0))],
        out_specs=pl.BlockSpec((KCOLS, 2 * d), lambda j: (j, 0)),
        out_shape=jax.ShapeDtypeStruct((vpad, 2 * d), jnp.float32),
    )


def _make_gather(B, D):
    n_per_w = B // NW
    n_chunks = n_per_w // CHUNK
    mesh = plsc.VectorSubcoreMesh(core_axis_name="c", subcore_axis_name="s")

    @functools.partial(
        pl.kernel,
        mesh=mesh,
        out_type=jax.ShapeDtypeStruct((B, D), jnp.float32),
        compiler_params=pltpu.CompilerParams(needs_layout_passes=False),
        scratch_types=[
            pltpu.VMEM((n_per_w,), jnp.int32),
            pltpu.VMEM((2, CHUNK, 2 * D), jnp.float32),
            pltpu.VMEM((2, CHUNK, D), jnp.float32),
            pltpu.SemaphoreType.DMA,
            pltpu.SemaphoreType.DMA,
            pltpu.SemaphoreType.DMA,
            pltpu.SemaphoreType.DMA,
        ],
    )
    def k(xp_hbm, idx_hbm, out_hbm, idx_v, pairs_v, rows_v, *sems):
        g0, g1, s0, s1 = sems
        gsems = (g0, g1)
        ssems = (s0, s1)
        wid = lax.axis_index("s") * NC + lax.axis_index("c")
        row0 = wid * n_per_w
        pltpu.sync_copy(idx_hbm.at[pl.ds(row0, n_per_w)], idx_v)

        def fire(c, b):
            pltpu.async_copy(
                xp_hbm.at[idx_v.at[pl.ds(c * CHUNK, CHUNK)]],
                pairs_v.at[b], gsems[b])

        def drain_gather(b):
            pltpu.make_async_copy(
                xp_hbm.at[pl.ds(0, CHUNK)], pairs_v.at[b], gsems[b]).wait()

        def select_store(c, b):
            def group(g, carry):
                for j in range(L):
                    r = g * L + j
                    for jb in range(D // L):
                        rows_v[b, r, pl.ds(jb * L, L)] = (
                            pairs_v[b, r, pl.ds(jb * L, L)])
                return carry

            lax.fori_loop(0, CHUNK // L, group, 0)
            pltpu.async_copy(
                rows_v.at[b],
                out_hbm.at[pl.ds(row0 + c * CHUNK, CHUNK)], ssems[b])

        def drain_store(b):
            pltpu.make_async_copy(
                rows_v.at[b], out_hbm.at[pl.ds(0, CHUNK)], ssems[b]).wait()

        fire(0, 0)

        def body(i, carry):
            c = i * 2
            for b in range(2):

                @pl.when(c + b + 1 < n_chunks)
                def _():
                    @pl.when(c + b >= 1)
                    def _():
                        drain_store(1 - b)

                    fire(c + b + 1, 1 - b)

                drain_gather(b)
                select_store(c + b, b)
            return carry

        lax.fori_loop(0, n_chunks // 2, body, 0)
        drain_store(0)
        drain_store(1)

    return k


def kernel(x, dim, idx):
    B = idx.shape[0]
    V, D = x.shape
    idx32 = (idx + dim).astype(jnp.int32)
    xp = _pack_kernel(V, D)(x.T, jnp.eye(D, dtype=jnp.float32))
    return _make_gather(B, D)(xp, idx32)
